# Initial kernel scaffold; baseline (speedup 1.0000x reference)
import jax
import jax.numpy as jnp
from jax.experimental import pallas as pl


def kernel(x, A, temperature):
    raise NotImplementedError("write your pallas kernel here")


# speedup vs baseline: 8.5520x; 8.5520x over previous
import functools

import jax
import jax.numpy as jnp
from jax import lax
from jax.experimental import pallas as pl
from jax.experimental.pallas import tpu as pltpu

_N = 4096
_D = 256
_K = 8
_ROWS = 128
_GRID = _N // _ROWS


def _threefry_bits(j_u32):
    ks0 = jnp.uint32(0)
    ks1 = jnp.uint32(42)
    ks2 = jnp.uint32(0x1BD11BDA) ^ ks0 ^ ks1
    ks = (ks0, ks1, ks2)
    rot = ((13, 15, 26, 6), (17, 29, 16, 24))
    x0 = jnp.zeros_like(j_u32) + ks0
    x1 = j_u32 + ks1
    for i in range(5):
        for r in rot[i % 2]:
            x0 = x0 + x1
            x1 = lax.shift_left(x1, jnp.uint32(r)) | lax.shift_right_logical(
                x1, jnp.uint32(32 - r))
            x1 = x1 ^ x0
        x0 = x0 + ks[(i + 1) % 3]
        x1 = x1 + ks[(i + 2) % 3] + jnp.uint32(i + 1)
    return x0 ^ x1


def _body(t_ref, xb_ref, xt_ref, vals_ref, idx_ref):
    i = pl.program_id(0)
    xb = xb_ref[...]
    xt = xt_ref[...]

    g = jax.lax.dot_general(xb, xt, (((1,), (0,)), ((), ())),
                            preferred_element_type=jnp.float32)
    sq_r = jnp.sum(xb * xb, axis=1, keepdims=True)
    sq_c = jnp.sum(xt * xt, axis=0, keepdims=True)
    dist = jnp.maximum((sq_r + sq_c) - 2.0 * g, 0.0)

    row = lax.broadcasted_iota(jnp.int32, (_ROWS, _N), 0)
    col = lax.broadcasted_iota(jnp.int32, (_ROWS, _N), 1)
    j = ((i * _ROWS + row) * _N + col).astype(jnp.uint32)
    bits = _threefry_bits(j)
    fb = lax.shift_right_logical(bits, jnp.uint32(9)) | jnp.uint32(0x3F800000)
    u = jnp.maximum(lax.bitcast_convert_type(fb, jnp.float32) - 1.0, 0.0)
    q = u + 1e-8

    s = jnp.exp(jnp.clip(t_ref[0], -5.0, 5.0))
    score = jnp.log(-jnp.log(q)) - dist * s

    big = jnp.int32(_N)
    neg_inf = jnp.float32(-jnp.inf)
    sc = score
    for k in range(_K):
        m = jnp.max(sc, axis=1, keepdims=True)
        am = jnp.min(jnp.where(sc >= m, col, big), axis=1, keepdims=True)
        vals_ref[:, k:k + 1] = m
        idx_ref[:, k:k + 1] = am
        if k + 1 < _K:
            sc = jnp.where(col == am, neg_inf, sc)


@jax.jit
def kernel(x, A, temperature):
    del A
    x2d = x[0] if x.ndim == 3 else x
    xt = x2d.T
    t_arr = jnp.reshape(temperature.astype(jnp.float32), (1,))

    vals, idx = pl.pallas_call(
        _body,
        grid=(_GRID,),
        in_specs=[
            pl.BlockSpec(memory_space=pltpu.SMEM),
            pl.BlockSpec((_ROWS, _D), lambda i: (i, 0)),
            pl.BlockSpec((_D, _N), lambda i: (0, 0)),
        ],
        out_specs=[
            pl.BlockSpec((_ROWS, _K), lambda i: (i, 0)),
            pl.BlockSpec((_ROWS, _K), lambda i: (i, 0)),
        ],
        out_shape=[
            jax.ShapeDtypeStruct((_N, _K), jnp.float32),
            jax.ShapeDtypeStruct((_N, _K), jnp.int32),
        ],
        compiler_params=pltpu.CompilerParams(
            dimension_semantics=("arbitrary",),
        ),
    )(t_arr, x2d, xt)

    rows = jnp.broadcast_to(jnp.arange(_N, dtype=jnp.int32)[:, None],
                            (_N, _K))
    edges_hat = jnp.stack([idx.reshape(-1), rows.reshape(-1)], axis=0)
    x_out = x2d[None] if x.ndim != 3 else x
    return (x_out, edges_hat, vals[None])


# speedup vs baseline: 9.1000x; 1.0641x over previous
import functools

import jax
import jax.numpy as jnp
from jax import lax
from jax.experimental import pallas as pl
from jax.experimental.pallas import tpu as pltpu

_N = 4096
_D = 256
_K = 8
_ROWS = 256
_GRID = _N // _ROWS


_KS0 = 0
_KS1 = 42
_KS2 = 0x1BD11BDA ^ _KS0 ^ _KS1
_KS = (_KS0, _KS1, _KS2)


def _threefry_bits(j_u32):
    rot = ((13, 15, 26, 6), (17, 29, 16, 24))

    def rotl(v, r):
        return lax.shift_left(v, jnp.uint32(r)) | lax.shift_right_logical(
            v, jnp.uint32(32 - r))

    x1 = j_u32 + jnp.uint32(_KS1)
    x0 = x1
    t = rotl(x1, rot[0][0])
    x1 = t ^ x0
    first = True
    for i in range(5):
        for r in rot[i % 2]:
            if first:
                first = False
                continue
            x0 = x0 + x1
            x1 = rotl(x1, r) ^ x0
        x0 = x0 + jnp.uint32(_KS[(i + 1) % 3])
        x1 = x1 + jnp.uint32((_KS[(i + 2) % 3] + i + 1) & 0xFFFFFFFF)
    return x0 ^ x1


def _body(t_ref, xb_ref, xt_ref, vals_ref, idx_ref):
    i = pl.program_id(0)
    xb = xb_ref[...]
    xt = xt_ref[...]

    g = jax.lax.dot_general(xb, xt, (((1,), (0,)), ((), ())),
                            preferred_element_type=jnp.float32)
    sq_r = jnp.sum(xb * xb, axis=1, keepdims=True)
    sq_c = jnp.sum(xt * xt, axis=0, keepdims=True)
    dist = jnp.maximum((sq_r + sq_c) - 2.0 * g, 0.0)

    row = lax.broadcasted_iota(jnp.int32, (_ROWS, _N), 0)
    col = lax.broadcasted_iota(jnp.int32, (_ROWS, _N), 1)
    j = ((i * _ROWS + row) * _N + col).astype(jnp.uint32)
    bits = _threefry_bits(j)
    fb = lax.shift_right_logical(bits, jnp.uint32(9)) | jnp.uint32(0x3F800000)
    u = jnp.maximum(lax.bitcast_convert_type(fb, jnp.float32) - 1.0, 0.0)
    q = u + 1e-8

    s = jnp.exp(jnp.clip(t_ref[0], -5.0, 5.0))
    score = jnp.log(-jnp.log(q)) - dist * s

    big = jnp.int32(_N)
    neg_inf = jnp.float32(-jnp.inf)
    sc = score
    for k in range(_K):
        m = jnp.max(sc, axis=1, keepdims=True)
        am = jnp.min(jnp.where(sc >= m, col, big), axis=1, keepdims=True)
        vals_ref[:, k:k + 1] = m
        idx_ref[:, k:k + 1] = am
        if k + 1 < _K:
            sc = jnp.where(col == am, neg_inf, sc)


@jax.jit
def kernel(x, A, temperature):
    del A
    x2d = x[0] if x.ndim == 3 else x
    xt = x2d.T
    t_arr = jnp.reshape(temperature.astype(jnp.float32), (1,))

    vals, idx = pl.pallas_call(
        _body,
        grid=(_GRID,),
        in_specs=[
            pl.BlockSpec(memory_space=pltpu.SMEM),
            pl.BlockSpec((_ROWS, _D), lambda i: (i, 0)),
            pl.BlockSpec((_D, _N), lambda i: (0, 0)),
        ],
        out_specs=[
            pl.BlockSpec((_ROWS, _K), lambda i: (i, 0)),
            pl.BlockSpec((_ROWS, _K), lambda i: (i, 0)),
        ],
        out_shape=[
            jax.ShapeDtypeStruct((_N, _K), jnp.float32),
            jax.ShapeDtypeStruct((_N, _K), jnp.int32),
        ],
        compiler_params=pltpu.CompilerParams(
            dimension_semantics=("arbitrary",),
        ),
    )(t_arr, x2d, xt)

    rows = jnp.broadcast_to(jnp.arange(_N, dtype=jnp.int32)[:, None],
                            (_N, _K))
    edges_hat = jnp.stack([idx.reshape(-1), rows.reshape(-1)], axis=0)
    x_out = x2d[None] if x.ndim != 3 else x
    return (x_out, edges_hat, vals[None])


# speedup vs baseline: 9.2289x; 1.0142x over previous
import functools

import jax
import jax.numpy as jnp
from jax import lax
from jax.experimental import pallas as pl
from jax.experimental.pallas import tpu as pltpu
from jax.experimental.pallas import tpu_sc as plsc

_N = 4096
_D = 256
_K = 8
_ROWS = 256
_R_SC = 1024
_R_TC = _N - _R_SC

_KS0 = 0
_KS1 = 42
_KS2 = 0x1BD11BDA ^ _KS0 ^ _KS1
_KS = (_KS0, _KS1, _KS2)


def _threefry_bits(j_u32):
    rot = ((13, 15, 26, 6), (17, 29, 16, 24))

    def rotl(v, r):
        return lax.shift_left(v, jnp.uint32(r)) | lax.shift_right_logical(
            v, jnp.uint32(32 - r))

    x1 = j_u32 + jnp.uint32(_KS1)
    x0 = x1
    x1 = rotl(x1, rot[0][0]) ^ x0
    first = True
    for i in range(5):
        for r in rot[i % 2]:
            if first:
                first = False
                continue
            x0 = x0 + x1
            x1 = rotl(x1, r) ^ x0
        x0 = x0 + jnp.uint32(_KS[(i + 1) % 3])
        x1 = x1 + jnp.uint32((_KS[(i + 2) % 3] + i + 1) & 0xFFFFFFFF)
    return x0 ^ x1


_SC_NW = 32
_SC_ROWS_PER_W = _R_SC // _SC_NW
_SC_CHUNK_ROWS = 4
_SC_CHUNKS = _SC_ROWS_PER_W // _SC_CHUNK_ROWS
_SC_CHUNK_ELEMS = _SC_CHUNK_ROWS * _N
_SC_UNROLL = 4


def _sc_bits_body(out_hbm, buf):
    nc = 2
    wid = lax.axis_index("s") * nc + lax.axis_index("c")
    iota = lax.iota(jnp.int32, 16)
    w_base = wid * (_SC_ROWS_PER_W * _N)
    for ch in range(_SC_CHUNKS):
        ch_base = w_base + ch * _SC_CHUNK_ELEMS

        @pl.loop(0, _SC_CHUNK_ELEMS // (16 * _SC_UNROLL))
        def _vec(t):
            for u in range(_SC_UNROLL):
                off = t * (16 * _SC_UNROLL) + u * 16
                j = ((_R_TC * _N) + ch_base + off + iota).astype(jnp.uint32)
                buf[pl.ds(off, 16)] = _threefry_bits(j)

        pltpu.sync_copy(buf, out_hbm.at[pl.ds(ch_base, _SC_CHUNK_ELEMS)])


_sc_bits = functools.partial(
    pl.kernel,
    mesh=plsc.VectorSubcoreMesh(core_axis_name="c", subcore_axis_name="s"),
    out_type=jax.ShapeDtypeStruct((_R_SC * _N,), jnp.uint32),
    scratch_types=[pltpu.VMEM((_SC_CHUNK_ELEMS,), jnp.uint32)],
)(_sc_bits_body)


def _scores_topk(t_ref, xb_ref, xt_ref, sqr_ref, sqc_ref, bits, vals_ref,
                 idx_ref, col):
    xb = xb_ref[...]
    xt = xt_ref[...]

    g = jax.lax.dot_general(xb, xt, (((1,), (0,)), ((), ())),
                            preferred_element_type=jnp.float32)
    sq_r = sqr_ref[...]
    sq_c = sqc_ref[...]
    dist = jnp.maximum((sq_r + sq_c) - 2.0 * g, 0.0)

    fb = lax.shift_right_logical(bits, jnp.uint32(9)) | jnp.uint32(0x3F800000)
    u = jnp.maximum(lax.bitcast_convert_type(fb, jnp.float32) - 1.0, 0.0)
    q = u + 1e-8

    s = jnp.exp(jnp.clip(t_ref[0], -5.0, 5.0))
    score = jnp.log(-jnp.log(q)) - dist * s

    big = jnp.int32(_N)
    neg_inf = jnp.float32(-jnp.inf)
    sc = score
    for k in range(_K):
        m = jnp.max(sc, axis=1, keepdims=True)
        am = jnp.min(jnp.where(sc >= m, col, big), axis=1, keepdims=True)
        vals_ref[:, k:k + 1] = m
        idx_ref[:, k:k + 1] = am
        if k + 1 < _K:
            sc = jnp.where(col == am, neg_inf, sc)


def _body_tc(t_ref, xb_ref, xt_ref, sqr_ref, sqc_ref, vals_ref, idx_ref):
    i = pl.program_id(0)
    row = lax.broadcasted_iota(jnp.int32, (_ROWS, _N), 0)
    col = lax.broadcasted_iota(jnp.int32, (_ROWS, _N), 1)
    j = ((i * _ROWS + row) * _N + col).astype(jnp.uint32)
    bits = _threefry_bits(j)
    _scores_topk(t_ref, xb_ref, xt_ref, sqr_ref, sqc_ref, bits, vals_ref,
                 idx_ref, col)


def _body_scbits(t_ref, xb_ref, xt_ref, sqr_ref, sqc_ref, bits_ref, vals_ref,
                 idx_ref):
    col = lax.broadcasted_iota(jnp.int32, (_ROWS, _N), 1)
    _scores_topk(t_ref, xb_ref, xt_ref, sqr_ref, sqc_ref, bits_ref[...],
                 vals_ref, idx_ref, col)


@jax.jit
def kernel(x, A, temperature):
    del A
    x2d = x[0] if x.ndim == 3 else x
    xt = x2d.T
    t_arr = jnp.reshape(temperature.astype(jnp.float32), (1,))
    sq = jnp.sum(x2d * x2d, axis=-1)
    sq_r = sq[:, None]
    sq_c = sq[None, :]

    bits_hi = _sc_bits().reshape(_R_SC, _N)

    vals_lo, idx_lo = pl.pallas_call(
        _body_tc,
        grid=(_R_TC // _ROWS,),
        in_specs=[
            pl.BlockSpec(memory_space=pltpu.SMEM),
            pl.BlockSpec((_ROWS, _D), lambda i: (i, 0)),
            pl.BlockSpec((_D, _N), lambda i: (0, 0)),
            pl.BlockSpec((_ROWS, 1), lambda i: (i, 0)),
            pl.BlockSpec((1, _N), lambda i: (0, 0)),
        ],
        out_specs=[
            pl.BlockSpec((_ROWS, _K), lambda i: (i, 0)),
            pl.BlockSpec((_ROWS, _K), lambda i: (i, 0)),
        ],
        out_shape=[
            jax.ShapeDtypeStruct((_R_TC, _K), jnp.float32),
            jax.ShapeDtypeStruct((_R_TC, _K), jnp.int32),
        ],
        compiler_params=pltpu.CompilerParams(
            dimension_semantics=("arbitrary",),
        ),
    )(t_arr, x2d[:_R_TC], xt, sq_r[:_R_TC], sq_c)

    vals_hi, idx_hi = pl.pallas_call(
        _body_scbits,
        grid=(_R_SC // _ROWS,),
        in_specs=[
            pl.BlockSpec(memory_space=pltpu.SMEM),
            pl.BlockSpec((_ROWS, _D), lambda i: (i, 0)),
            pl.BlockSpec((_D, _N), lambda i: (0, 0)),
            pl.BlockSpec((_ROWS, 1), lambda i: (i, 0)),
            pl.BlockSpec((1, _N), lambda i: (0, 0)),
            pl.BlockSpec((_ROWS, _N), lambda i: (i, 0)),
        ],
        out_specs=[
            pl.BlockSpec((_ROWS, _K), lambda i: (i, 0)),
            pl.BlockSpec((_ROWS, _K), lambda i: (i, 0)),
        ],
        out_shape=[
            jax.ShapeDtypeStruct((_R_SC, _K), jnp.float32),
            jax.ShapeDtypeStruct((_R_SC, _K), jnp.int32),
        ],
        compiler_params=pltpu.CompilerParams(
            dimension_semantics=("arbitrary",),
        ),
    )(t_arr, x2d[_R_TC:], xt, sq_r[_R_TC:], sq_c, bits_hi)

    vals = jnp.concatenate([vals_lo, vals_hi], axis=0)
    idx = jnp.concatenate([idx_lo, idx_hi], axis=0)
    rows = jnp.broadcast_to(jnp.arange(_N, dtype=jnp.int32)[:, None],
                            (_N, _K))
    edges_hat = jnp.stack([idx.reshape(-1), rows.reshape(-1)], axis=0)
    x_out = x2d[None] if x.ndim != 3 else x
    return (x_out, edges_hat, vals[None])


# speedup vs baseline: 9.6860x; 1.0495x over previous
import functools

import jax
import jax.numpy as jnp
from jax import lax
from jax.experimental import pallas as pl
from jax.experimental.pallas import tpu as pltpu
from jax.experimental.pallas import tpu_sc as plsc

_N = 4096
_D = 256
_K = 8
_ROWS = 256
_R_SC = 1536
_R_TC = _N - _R_SC

_KS0 = 0
_KS1 = 42
_KS2 = 0x1BD11BDA ^ _KS0 ^ _KS1
_KS = (_KS0, _KS1, _KS2)


def _threefry_bits(j_u32):
    rot = ((13, 15, 26, 6), (17, 29, 16, 24))

    def rotl(v, r):
        return lax.shift_left(v, jnp.uint32(r)) | lax.shift_right_logical(
            v, jnp.uint32(32 - r))

    x1 = j_u32 + jnp.uint32(_KS1)
    x0 = x1
    x1 = rotl(x1, rot[0][0]) ^ x0
    first = True
    for i in range(5):
        for r in rot[i % 2]:
            if first:
                first = False
                continue
            x0 = x0 + x1
            x1 = rotl(x1, r) ^ x0
        x0 = x0 + jnp.uint32(_KS[(i + 1) % 3])
        x1 = x1 + jnp.uint32((_KS[(i + 2) % 3] + i + 1) & 0xFFFFFFFF)
    return x0 ^ x1


_SC_NW = 32
_SC_ROWS_PER_W = _R_SC // _SC_NW
_SC_CHUNK_ROWS = 4
_SC_CHUNKS = _SC_ROWS_PER_W // _SC_CHUNK_ROWS
_SC_CHUNK_ELEMS = _SC_CHUNK_ROWS * _N
_SC_UNROLL = 8


def _sc_bits_body(out_hbm, buf):
    nc = 2
    wid = lax.axis_index("s") * nc + lax.axis_index("c")
    iota = lax.iota(jnp.int32, 16)
    w_base = wid * (_SC_ROWS_PER_W * _N)

    @pl.loop(0, _SC_CHUNKS)
    def _chunk(ch):
        ch_base = w_base + ch * _SC_CHUNK_ELEMS

        @pl.loop(0, _SC_CHUNK_ELEMS // (16 * _SC_UNROLL))
        def _vec(t):
            for u in range(_SC_UNROLL):
                off = t * (16 * _SC_UNROLL) + u * 16
                j = ((_R_TC * _N) + ch_base + off + iota).astype(jnp.uint32)
                buf[pl.ds(off, 16)] = _threefry_bits(j)

        pltpu.sync_copy(buf, out_hbm.at[pl.ds(ch_base, _SC_CHUNK_ELEMS)])


_sc_bits = functools.partial(
    pl.kernel,
    mesh=plsc.VectorSubcoreMesh(core_axis_name="c", subcore_axis_name="s"),
    out_type=jax.ShapeDtypeStruct((_R_SC * _N,), jnp.uint32),
    scratch_types=[pltpu.VMEM((_SC_CHUNK_ELEMS,), jnp.uint32)],
)(_sc_bits_body)


def _scores_topk(t_ref, xb_ref, xt_ref, sqr_ref, sqc_ref, bits, vals_ref,
                 idx_ref, col):
    xb = xb_ref[...]
    xt = xt_ref[...]

    g = jax.lax.dot_general(xb, xt, (((1,), (0,)), ((), ())),
                            preferred_element_type=jnp.float32)
    sq_r = sqr_ref[...]
    sq_c = sqc_ref[...]
    dist = jnp.maximum((sq_r + sq_c) - 2.0 * g, 0.0)

    fb = lax.shift_right_logical(bits, jnp.uint32(9)) | jnp.uint32(0x3F800000)
    u = jnp.maximum(lax.bitcast_convert_type(fb, jnp.float32) - 1.0, 0.0)
    q = u + 1e-8

    s = jnp.exp(jnp.clip(t_ref[0], -5.0, 5.0))
    score = jnp.log(-jnp.log(q)) - dist * s

    big = jnp.int32(_N)
    neg_inf = jnp.float32(-jnp.inf)
    sc = score
    for k in range(_K):
        m = jnp.max(sc, axis=1, keepdims=True)
        am = jnp.min(jnp.where(sc >= m, col, big), axis=1, keepdims=True)
        vals_ref[:, k:k + 1] = m
        idx_ref[:, k:k + 1] = am
        if k + 1 < _K:
            sc = jnp.where(col == am, neg_inf, sc)


def _body_tc(t_ref, xb_ref, xt_ref, sqr_ref, sqc_ref, vals_ref, idx_ref):
    i = pl.program_id(0)
    row = lax.broadcasted_iota(jnp.int32, (_ROWS, _N), 0)
    col = lax.broadcasted_iota(jnp.int32, (_ROWS, _N), 1)
    j = ((i * _ROWS + row) * _N + col).astype(jnp.uint32)
    bits = _threefry_bits(j)
    _scores_topk(t_ref, xb_ref, xt_ref, sqr_ref, sqc_ref, bits, vals_ref,
                 idx_ref, col)


def _body_scbits(t_ref, xb_ref, xt_ref, sqr_ref, sqc_ref, bits_ref, vals_ref,
                 idx_ref):
    col = lax.broadcasted_iota(jnp.int32, (_ROWS, _N), 1)
    _scores_topk(t_ref, xb_ref, xt_ref, sqr_ref, sqc_ref, bits_ref[...],
                 vals_ref, idx_ref, col)


@jax.jit
def kernel(x, A, temperature):
    del A
    x2d = x[0] if x.ndim == 3 else x
    xt = x2d.T
    t_arr = jnp.reshape(temperature.astype(jnp.float32), (1,))
    sq = jnp.sum(x2d * x2d, axis=-1)
    sq_r = sq[:, None]
    sq_c = sq[None, :]

    bits_hi = _sc_bits().reshape(_R_SC, _N)

    vals_lo, idx_lo = pl.pallas_call(
        _body_tc,
        grid=(_R_TC // _ROWS,),
        in_specs=[
            pl.BlockSpec(memory_space=pltpu.SMEM),
            pl.BlockSpec((_ROWS, _D), lambda i: (i, 0)),
            pl.BlockSpec((_D, _N), lambda i: (0, 0)),
            pl.BlockSpec((_ROWS, 1), lambda i: (i, 0)),
            pl.BlockSpec((1, _N), lambda i: (0, 0)),
        ],
        out_specs=[
            pl.BlockSpec((_ROWS, _K), lambda i: (i, 0)),
            pl.BlockSpec((_ROWS, _K), lambda i: (i, 0)),
        ],
        out_shape=[
            jax.ShapeDtypeStruct((_R_TC, _K), jnp.float32),
            jax.ShapeDtypeStruct((_R_TC, _K), jnp.int32),
        ],
        compiler_params=pltpu.CompilerParams(
            dimension_semantics=("arbitrary",),
        ),
    )(t_arr, x2d[:_R_TC], xt, sq_r[:_R_TC], sq_c)

    vals_hi, idx_hi = pl.pallas_call(
        _body_scbits,
        grid=(_R_SC // _ROWS,),
        in_specs=[
            pl.BlockSpec(memory_space=pltpu.SMEM),
            pl.BlockSpec((_ROWS, _D), lambda i: (i, 0)),
            pl.BlockSpec((_D, _N), lambda i: (0, 0)),
            pl.BlockSpec((_ROWS, 1), lambda i: (i, 0)),
            pl.BlockSpec((1, _N), lambda i: (0, 0)),
            pl.BlockSpec((_ROWS, _N), lambda i: (i, 0)),
        ],
        out_specs=[
            pl.BlockSpec((_ROWS, _K), lambda i: (i, 0)),
            pl.BlockSpec((_ROWS, _K), lambda i: (i, 0)),
        ],
        out_shape=[
            jax.ShapeDtypeStruct((_R_SC, _K), jnp.float32),
            jax.ShapeDtypeStruct((_R_SC, _K), jnp.int32),
        ],
        compiler_params=pltpu.CompilerParams(
            dimension_semantics=("arbitrary",),
        ),
    )(t_arr, x2d[_R_TC:], xt, sq_r[_R_TC:], sq_c, bits_hi)

    vals = jnp.concatenate([vals_lo, vals_hi], axis=0)
    idx = jnp.concatenate([idx_lo, idx_hi], axis=0)
    rows = jnp.broadcast_to(jnp.arange(_N, dtype=jnp.int32)[:, None],
                            (_N, _K))
    edges_hat = jnp.stack([idx.reshape(-1), rows.reshape(-1)], axis=0)
    x_out = x2d[None] if x.ndim != 3 else x
    return (x_out, edges_hat, vals[None])


# speedup vs baseline: 9.7557x; 1.0072x over previous
import functools

import jax
import jax.numpy as jnp
from jax import lax
from jax.experimental import pallas as pl
from jax.experimental.pallas import tpu as pltpu
from jax.experimental.pallas import tpu_sc as plsc

_N = 4096
_D = 256
_K = 8
_ROWS = 256
_R_SC = 1536
_R_TC = _N - _R_SC

_KS0 = 0
_KS1 = 42
_KS2 = 0x1BD11BDA ^ _KS0 ^ _KS1
_KS = (_KS0, _KS1, _KS2)


def _threefry_bits(j_u32):
    rot = ((13, 15, 26, 6), (17, 29, 16, 24))

    def rotl(v, r):
        return lax.shift_left(v, jnp.uint32(r)) | lax.shift_right_logical(
            v, jnp.uint32(32 - r))

    x1 = j_u32 + jnp.uint32(_KS1)
    x0 = x1
    x1 = rotl(x1, rot[0][0]) ^ x0
    first = True
    for i in range(5):
        for r in rot[i % 2]:
            if first:
                first = False
                continue
            x0 = x0 + x1
            x1 = rotl(x1, r) ^ x0
        x0 = x0 + jnp.uint32(_KS[(i + 1) % 3])
        x1 = x1 + jnp.uint32((_KS[(i + 2) % 3] + i + 1) & 0xFFFFFFFF)
    return x0 ^ x1


_SC_NW = 32
_SC_ROWS_PER_W = _R_SC // _SC_NW
_SC_CHUNK_ROWS = 4
_SC_CHUNKS = _SC_ROWS_PER_W // _SC_CHUNK_ROWS
_SC_CHUNK_ELEMS = _SC_CHUNK_ROWS * _N
_SC_UNROLL = 16


def _sc_bits_body(out_hbm, buf):
    nc = 2
    wid = lax.axis_index("s") * nc + lax.axis_index("c")
    iota = lax.iota(jnp.int32, 16)
    w_base = wid * (_SC_ROWS_PER_W * _N)

    @pl.loop(0, _SC_CHUNKS)
    def _chunk(ch):
        ch_base = w_base + ch * _SC_CHUNK_ELEMS

        @pl.loop(0, _SC_CHUNK_ELEMS // (16 * _SC_UNROLL))
        def _vec(t):
            for u in range(_SC_UNROLL):
                off = t * (16 * _SC_UNROLL) + u * 16
                j = ((_R_TC * _N) + ch_base + off + iota).astype(jnp.uint32)
                buf[pl.ds(off, 16)] = _threefry_bits(j)

        pltpu.sync_copy(buf, out_hbm.at[pl.ds(ch_base, _SC_CHUNK_ELEMS)])


_sc_bits = functools.partial(
    pl.kernel,
    mesh=plsc.VectorSubcoreMesh(core_axis_name="c", subcore_axis_name="s"),
    out_type=jax.ShapeDtypeStruct((_R_SC * _N,), jnp.uint32),
    scratch_types=[pltpu.VMEM((_SC_CHUNK_ELEMS,), jnp.uint32)],
)(_sc_bits_body)


def _scores_topk(t_ref, xb_ref, xt_ref, sqr_ref, sqc_ref, bits, vals_ref,
                 idx_ref, col):
    xb = xb_ref[...]
    xt = xt_ref[...]

    g = jax.lax.dot_general(xb, xt, (((1,), (0,)), ((), ())),
                            preferred_element_type=jnp.float32)
    sq_r = sqr_ref[...]
    sq_c = sqc_ref[...]
    dist = jnp.maximum((sq_r + sq_c) - 2.0 * g, 0.0)

    fb = lax.shift_right_logical(bits, jnp.uint32(9)) | jnp.uint32(0x3F800000)
    u = jnp.maximum(lax.bitcast_convert_type(fb, jnp.float32) - 1.0, 0.0)
    q = u + 1e-8

    s = jnp.exp(jnp.clip(t_ref[0], -5.0, 5.0))
    score = jnp.log(-jnp.log(q)) - dist * s

    big = jnp.int32(_N)
    neg_inf = jnp.float32(-jnp.inf)
    sc = score
    for k in range(_K):
        m = jnp.max(sc, axis=1, keepdims=True)
        am = jnp.min(jnp.where(sc >= m, col, big), axis=1, keepdims=True)
        vals_ref[:, k:k + 1] = m
        idx_ref[:, k:k + 1] = am
        if k + 1 < _K:
            sc = jnp.where(col == am, neg_inf, sc)


def _body_tc(t_ref, xb_ref, xt_ref, sqr_ref, sqc_ref, vals_ref, idx_ref):
    i = pl.program_id(0)
    row = lax.broadcasted_iota(jnp.int32, (_ROWS, _N), 0)
    col = lax.broadcasted_iota(jnp.int32, (_ROWS, _N), 1)
    j = ((i * _ROWS + row) * _N + col).astype(jnp.uint32)
    bits = _threefry_bits(j)
    _scores_topk(t_ref, xb_ref, xt_ref, sqr_ref, sqc_ref, bits, vals_ref,
                 idx_ref, col)


def _body_scbits(t_ref, xb_ref, xt_ref, sqr_ref, sqc_ref, bits_ref, vals_ref,
                 idx_ref):
    col = lax.broadcasted_iota(jnp.int32, (_ROWS, _N), 1)
    _scores_topk(t_ref, xb_ref, xt_ref, sqr_ref, sqc_ref, bits_ref[...],
                 vals_ref, idx_ref, col)


@jax.jit
def kernel(x, A, temperature):
    del A
    x2d = x[0] if x.ndim == 3 else x
    xt = x2d.T
    t_arr = jnp.reshape(temperature.astype(jnp.float32), (1,))
    sq = jnp.sum(x2d * x2d, axis=-1)
    sq_r = sq[:, None]
    sq_c = sq[None, :]

    bits_hi = _sc_bits().reshape(_R_SC, _N)

    vals_lo, idx_lo = pl.pallas_call(
        _body_tc,
        grid=(_R_TC // _ROWS,),
        in_specs=[
            pl.BlockSpec(memory_space=pltpu.SMEM),
            pl.BlockSpec((_ROWS, _D), lambda i: (i, 0)),
            pl.BlockSpec((_D, _N), lambda i: (0, 0)),
            pl.BlockSpec((_ROWS, 1), lambda i: (i, 0)),
            pl.BlockSpec((1, _N), lambda i: (0, 0)),
        ],
        out_specs=[
            pl.BlockSpec((_ROWS, _K), lambda i: (i, 0)),
            pl.BlockSpec((_ROWS, _K), lambda i: (i, 0)),
        ],
        out_shape=[
            jax.ShapeDtypeStruct((_R_TC, _K), jnp.float32),
            jax.ShapeDtypeStruct((_R_TC, _K), jnp.int32),
        ],
        compiler_params=pltpu.CompilerParams(
            dimension_semantics=("arbitrary",),
        ),
    )(t_arr, x2d, xt, sq_r, sq_c)

    vals_hi, idx_hi = pl.pallas_call(
        _body_scbits,
        grid=(_R_SC // _ROWS,),
        in_specs=[
            pl.BlockSpec(memory_space=pltpu.SMEM),
            pl.BlockSpec((_ROWS, _D), lambda i: (i + _R_TC // _ROWS, 0)),
            pl.BlockSpec((_D, _N), lambda i: (0, 0)),
            pl.BlockSpec((_ROWS, 1), lambda i: (i + _R_TC // _ROWS, 0)),
            pl.BlockSpec((1, _N), lambda i: (0, 0)),
            pl.BlockSpec((_ROWS, _N), lambda i: (i, 0)),
        ],
        out_specs=[
            pl.BlockSpec((_ROWS, _K), lambda i: (i, 0)),
            pl.BlockSpec((_ROWS, _K), lambda i: (i, 0)),
        ],
        out_shape=[
            jax.ShapeDtypeStruct((_R_SC, _K), jnp.float32),
            jax.ShapeDtypeStruct((_R_SC, _K), jnp.int32),
        ],
        compiler_params=pltpu.CompilerParams(
            dimension_semantics=("arbitrary",),
        ),
    )(t_arr, x2d, xt, sq_r, sq_c, bits_hi)

    vals = jnp.concatenate([vals_lo, vals_hi], axis=0)
    idx = jnp.concatenate([idx_lo, idx_hi], axis=0)
    rows = jnp.broadcast_to(jnp.arange(_N, dtype=jnp.int32)[:, None],
                            (_N, _K))
    edges_hat = jnp.stack([idx.reshape(-1), rows.reshape(-1)], axis=0)
    x_out = x2d[None] if x.ndim != 3 else x
    return (x_out, edges_hat, vals[None])
